# baseline (speedup 1.0000x reference)
import functools

import jax
import jax.numpy as jnp
from jax import lax
from jax.experimental import pallas as pl
from jax.experimental.pallas import tpu as pltpu
from jax.experimental.pallas import tpu_sc as plsc

D = 64
L = 16
EPS = 1e-5
NBUF = 2
UNROLL = 4


def _splat_sum(v):
    iota = lax.broadcasted_iota(jnp.int32, (L,), 0)
    for sh in (8, 4, 2, 1):
        v = v + v.at[iota ^ sh].get(mode="promise_in_bounds")
    return v


def _rsqrt(x):
    i = lax.bitcast_convert_type(x, jnp.int32)
    i = jnp.int32(0x5F3759DF) - (i >> 1)
    y = lax.bitcast_convert_type(i, jnp.float32)
    xh = x * 0.5
    y = y * (1.5 - xh * y * y)
    y = y * (1.5 - xh * y * y)
    return y


def _ln_row(data, r, g, bt):
    x = [data[r, pl.ds(j * L, L)] for j in range(D // L)]
    s = (x[0] + x[1]) + (x[2] + x[3])
    q = ((x[0] * x[0] + x[1] * x[1]) + (x[2] * x[2] + x[3] * x[3]))
    mean = _splat_sum(s) * (1.0 / D)
    ex2 = _splat_sum(q) * (1.0 / D) + EPS
    rstd = _rsqrt(ex2 - mean * mean)
    for j in range(D // L):
        data[r, pl.ds(j * L, L)] = (x[j] - mean) * rstd * g[j] + bt[j]


@jax.jit
def _transpose_table(table_t):
    V = table_t.shape[1]
    NC, NS = 2, 16
    NW = NC * NS
    n_blocks = (V + 127) // 128
    base_blocks = n_blocks // NW
    extra = n_blocks - base_blocks * NW

    mesh = plsc.VectorSubcoreMesh(core_axis_name="c", subcore_axis_name="s")

    @functools.partial(
        pl.kernel,
        mesh=mesh,
        out_type=jax.ShapeDtypeStruct((n_blocks * 128 * D,), jnp.float32),
        scratch_types=[
            pltpu.VMEM((D, 128), jnp.float32),
            pltpu.VMEM((D, 128), jnp.float32),
            pltpu.VMEM((128 * D,), jnp.float32),
            pltpu.VMEM((128 * D,), jnp.float32),
            pltpu.SemaphoreType.DMA((NBUF,)),
            pltpu.SemaphoreType.DMA((NBUF,)),
        ],
        compiler_params=pltpu.CompilerParams(
            use_tc_tiling_on_sc=True, needs_layout_passes=False),
    )
    def k(tt_hbm, tlin_hbm, in_v0, in_v1, out_v0, out_v1, isem, osem):
        in_v = [in_v0, in_v1]
        out_v = [out_v0, out_v1]
        wid = lax.axis_index("s") * NC + lax.axis_index("c")
        nblk = base_blocks + jnp.where(wid < extra, 1, 0)
        start = wid * base_blocks + jnp.minimum(wid, extra)

        def slab_in(i, b):
            return pltpu.make_async_copy(
                tt_hbm.at[:, pl.ds((start + i) * 128, 128)],
                in_v[b], isem.at[b])

        def slab_out(i, b):
            return pltpu.make_async_copy(
                out_v[b],
                tlin_hbm.at[pl.ds((start + i) * 128 * D, 128 * D)],
                osem.at[b])

        iota = lax.broadcasted_iota(jnp.int32, (L,), 0)
        iota64 = iota * D
        zeros = jnp.zeros((L,), jnp.int32)
        cols = [iota + j * L for j in range(8)]

        for b in range(NBUF):
            @pl.when(b < nblk)
            def _():
                slab_in(b, b).start()

        n_pairs = base_blocks // NBUF + 1

        def pair_body(g, carry):
            for b in range(NBUF):
                i = g * NBUF + b

                @pl.when(i < nblk)
                def _():
                    slab_in(i, b).wait()

                    @pl.when(i >= NBUF)
                    def _():
                        slab_out(i - NBUF, b).wait()

                    def f_body(f2, c2):
                        for u in range(4):
                            f = f2 * 4 + u
                            rowv = zeros + f
                            for j in range(8):
                                v = plsc.load_gather(in_v[b], [rowv, cols[j]])
                                addr = iota64 + (j * L * D + f)
                                plsc.store_scatter(out_v[b], [addr], v)
                        return c2

                    lax.fori_loop(0, D // 4, f_body, 0)
                    slab_out(i, b).start()

                    @pl.when(i + NBUF < nblk)
                    def _():
                        slab_in(i + NBUF, b).start()
            return carry

        lax.fori_loop(0, n_pairs, pair_body, 0)
        for b in range(NBUF):
            last = jnp.where(lax.rem(nblk - 1, NBUF) == b, nblk - 1, nblk - 2)

            @pl.when(last >= 0)
            def _():
                slab_out(last, b).wait()

    return k(table_t)


@functools.partial(jax.jit, static_argnames=("n_chunks", "chunk", "out_rows", "out_cols"))
def _embed_ln(table, idx, gamma, beta, n_chunks, chunk, out_rows, out_cols):
    B = idx.shape[0]
    NC, NS = 2, 16
    NW = NC * NS
    b_per_w = B // NW
    n_pairs = n_chunks // NBUF
    nb = chunk // out_cols

    mesh = plsc.VectorSubcoreMesh(core_axis_name="c", subcore_axis_name="s")

    @functools.partial(
        pl.kernel,
        mesh=mesh,
        out_type=jax.ShapeDtypeStruct((out_rows, out_cols, D), jnp.float32),
        scratch_types=[
            pltpu.VMEM((b_per_w,), jnp.int32),
            pltpu.VMEM((NBUF, chunk, D), jnp.float32),
            pltpu.VMEM((D,), jnp.float32),
            pltpu.VMEM((D,), jnp.float32),
            pltpu.SemaphoreType.DMA((NBUF,)),
            pltpu.SemaphoreType.DMA((NBUF,)),
        ],
        compiler_params=pltpu.CompilerParams(use_tc_tiling_on_sc=False),
    )
    def k(table_hbm, idx_hbm, gamma_hbm, beta_hbm, out_hbm,
          idx_v, data_v, g_v, bt_v, gsem, osem):
        wid = lax.axis_index("s") * NC + lax.axis_index("c")
        base = wid * b_per_w

        pltpu.sync_copy(idx_hbm.at[pl.ds(base, b_per_w)], idx_v)
        pltpu.sync_copy(gamma_hbm, g_v)
        pltpu.sync_copy(beta_hbm, bt_v)
        g = [g_v[pl.ds(j * L, L)] for j in range(D // L)]
        bt = [bt_v[pl.ds(j * L, L)] for j in range(D // L)]

        def gather(ci, b):
            return pltpu.make_async_copy(
                table_hbm.at[idx_v.at[pl.ds(ci * chunk, chunk)]],
                data_v.at[b], gsem.at[b])

        def writeback_copies(ci, b):
            blk0 = (base + ci * chunk) // out_cols
            return [
                pltpu.make_async_copy(
                    data_v.at[b, pl.ds(j * out_cols, out_cols)],
                    out_hbm.at[blk0 + j], osem.at[b])
                for j in range(nb)
            ]

        for b in range(NBUF):
            gather(b, b).start()

        def pair_body(gi, carry):
            for b in range(NBUF):
                ci = gi * NBUF + b
                gather(ci, b).wait()

                def rows_body(t, c2):
                    r0 = t * UNROLL
                    for u in range(UNROLL):
                        _ln_row(data_v.at[b], r0 + u, g, bt)
                    return c2

                lax.fori_loop(0, chunk // UNROLL, rows_body, 0)
                for c in writeback_copies(ci, b):
                    c.start()

                @pl.when(gi < n_pairs - 1)
                def _():
                    for c in writeback_copies(ci, b):
                        c.wait()
                    gather(ci + NBUF, b).start()
            return carry

        lax.fori_loop(0, n_pairs, pair_body, 0)
        for b in range(NBUF):
            for c in writeback_copies(n_chunks - NBUF + b, b):
                c.wait()

    return k(table, idx, gamma, beta)


def kernel(input, table, gamma, beta):
    idx = input.reshape(-1).astype(jnp.int32)
    B = idx.shape[0]
    chunk = 400
    n_chunks = B // (32 * chunk)
    tlin = _transpose_table(table.T).reshape(-1, D)
    return _embed_ln(tlin, idx, gamma, beta, n_chunks, chunk,
                     input.shape[0], input.shape[1])


# speedup vs baseline: 1.0959x; 1.0959x over previous
import functools

import jax
import jax.numpy as jnp
from jax import lax
from jax.experimental import pallas as pl
from jax.experimental.pallas import tpu as pltpu
from jax.experimental.pallas import tpu_sc as plsc

D = 64
L = 16
EPS = 1e-5
NBUF = 2
UNROLL = 4


def _splat_sum(v):
    iota = lax.broadcasted_iota(jnp.int32, (L,), 0)
    for sh in (8, 4, 2, 1):
        v = v + v.at[iota ^ sh].get(mode="promise_in_bounds")
    return v


def _rsqrt(x):
    i = lax.bitcast_convert_type(x, jnp.int32)
    i = jnp.int32(0x5F3759DF) - (i >> 1)
    y = lax.bitcast_convert_type(i, jnp.float32)
    xh = x * 0.5
    y = y * (1.5 - xh * y * y)
    y = y * (1.5 - xh * y * y)
    return y


def _ln_row(data, r, g, bt):
    x = [data[r, pl.ds(j * L, L)] for j in range(D // L)]
    s = (x[0] + x[1]) + (x[2] + x[3])
    q = ((x[0] * x[0] + x[1] * x[1]) + (x[2] * x[2] + x[3] * x[3]))
    mean = _splat_sum(s) * (1.0 / D)
    ex2 = _splat_sum(q) * (1.0 / D) + EPS
    rstd = _rsqrt(ex2 - mean * mean)
    for j in range(D // L):
        data[r, pl.ds(j * L, L)] = (x[j] - mean) * rstd * g[j] + bt[j]


@jax.jit
def _transpose_table(table_t):
    V = table_t.shape[1]
    NC, NS = 2, 16
    NW = NC * NS
    n_blocks = (V + 127) // 128
    base_blocks = n_blocks // NW
    extra = n_blocks - base_blocks * NW

    mesh = plsc.VectorSubcoreMesh(core_axis_name="c", subcore_axis_name="s")

    @functools.partial(
        pl.kernel,
        mesh=mesh,
        out_type=jax.ShapeDtypeStruct((n_blocks * 128 * D,), jnp.float32),
        scratch_types=[
            pltpu.VMEM((D, 128), jnp.float32),
            pltpu.VMEM((D, 128), jnp.float32),
            pltpu.VMEM((128 * D,), jnp.float32),
            pltpu.VMEM((128 * D,), jnp.float32),
            pltpu.VMEM((128 * (D + 1),), jnp.float32),
            pltpu.SemaphoreType.DMA((NBUF,)),
            pltpu.SemaphoreType.DMA((NBUF,)),
        ],
        compiler_params=pltpu.CompilerParams(
            use_tc_tiling_on_sc=True, needs_layout_passes=False),
    )
    def k(tt_hbm, tlin_hbm, in_v0, in_v1, out_v0, out_v1, skew_v, isem, osem):
        in_v = [in_v0, in_v1]
        out_v = [out_v0, out_v1]
        wid = lax.axis_index("s") * NC + lax.axis_index("c")
        nblk = base_blocks + jnp.where(wid < extra, 1, 0)
        start = wid * base_blocks + jnp.minimum(wid, extra)

        def slab_in(i, b):
            return pltpu.make_async_copy(
                tt_hbm.at[:, pl.ds((start + i) * 128, 128)],
                in_v[b], isem.at[b])

        def slab_out(i, b):
            return pltpu.make_async_copy(
                out_v[b],
                tlin_hbm.at[pl.ds((start + i) * 128 * D, 128 * D)],
                osem.at[b])

        S = D + 1
        iota = lax.broadcasted_iota(jnp.int32, (L,), 0)
        iota_skew = iota * S
        zeros = jnp.zeros((L,), jnp.int32)
        cols = [iota + j * L for j in range(8)]

        for b in range(NBUF):
            @pl.when(b < nblk)
            def _():
                slab_in(b, b).start()

        n_pairs = base_blocks // NBUF + 1

        def pair_body(g, carry):
            for b in range(NBUF):
                i = g * NBUF + b

                @pl.when(i < nblk)
                def _():
                    slab_in(i, b).wait()

                    @pl.when(i >= NBUF)
                    def _():
                        slab_out(i - NBUF, b).wait()

                    def f_body(f2, c2):
                        for u in range(2):
                            f = f2 * 2 + u
                            rowv = zeros + f
                            for j in range(8):
                                v = plsc.load_gather(in_v[b], [rowv, cols[j]])
                                addr = iota_skew + (j * L * S + f)
                                plsc.store_scatter(skew_v, [addr], v)
                        return c2

                    lax.fori_loop(0, D // 2, f_body, 0)

                    def e_body(e2, c2):
                        for u in range(2):
                            e = e2 * 2 + u
                            for j in range(D // L):
                                src = zeros + (e * S + j * L)
                                v = plsc.load_gather(skew_v, [src + iota])
                                out_v[b][pl.ds(e * D + j * L, L)] = v
                        return c2

                    lax.fori_loop(0, 64, e_body, 0)
                    slab_out(i, b).start()

                    @pl.when(i + NBUF < nblk)
                    def _():
                        slab_in(i + NBUF, b).start()
            return carry

        lax.fori_loop(0, n_pairs, pair_body, 0)
        for b in range(NBUF):
            last = jnp.where(lax.rem(nblk - 1, NBUF) == b, nblk - 1, nblk - 2)

            @pl.when(last >= 0)
            def _():
                slab_out(last, b).wait()

    return k(table_t)


@functools.partial(jax.jit, static_argnames=("n_chunks", "chunk", "out_rows", "out_cols"))
def _embed_ln(table, idx, gamma, beta, n_chunks, chunk, out_rows, out_cols):
    B = idx.shape[0]
    NC, NS = 2, 16
    NW = NC * NS
    b_per_w = B // NW
    n_pairs = n_chunks // NBUF
    nb = chunk // out_cols

    mesh = plsc.VectorSubcoreMesh(core_axis_name="c", subcore_axis_name="s")

    @functools.partial(
        pl.kernel,
        mesh=mesh,
        out_type=jax.ShapeDtypeStruct((out_rows, out_cols, D), jnp.float32),
        scratch_types=[
            pltpu.VMEM((b_per_w,), jnp.int32),
            pltpu.VMEM((NBUF, chunk, D), jnp.float32),
            pltpu.VMEM((D,), jnp.float32),
            pltpu.VMEM((D,), jnp.float32),
            pltpu.SemaphoreType.DMA((NBUF,)),
            pltpu.SemaphoreType.DMA((NBUF,)),
        ],
        compiler_params=pltpu.CompilerParams(use_tc_tiling_on_sc=False),
    )
    def k(table_hbm, idx_hbm, gamma_hbm, beta_hbm, out_hbm,
          idx_v, data_v, g_v, bt_v, gsem, osem):
        wid = lax.axis_index("s") * NC + lax.axis_index("c")
        base = wid * b_per_w

        pltpu.sync_copy(idx_hbm.at[pl.ds(base, b_per_w)], idx_v)
        pltpu.sync_copy(gamma_hbm, g_v)
        pltpu.sync_copy(beta_hbm, bt_v)
        g = [g_v[pl.ds(j * L, L)] for j in range(D // L)]
        bt = [bt_v[pl.ds(j * L, L)] for j in range(D // L)]

        def gather(ci, b):
            return pltpu.make_async_copy(
                table_hbm.at[idx_v.at[pl.ds(ci * chunk, chunk)]],
                data_v.at[b], gsem.at[b])

        def writeback_copies(ci, b):
            blk0 = (base + ci * chunk) // out_cols
            return [
                pltpu.make_async_copy(
                    data_v.at[b, pl.ds(j * out_cols, out_cols)],
                    out_hbm.at[blk0 + j], osem.at[b])
                for j in range(nb)
            ]

        for b in range(NBUF):
            gather(b, b).start()

        def pair_body(gi, carry):
            for b in range(NBUF):
                ci = gi * NBUF + b
                gather(ci, b).wait()

                def rows_body(t, c2):
                    r0 = t * UNROLL
                    for u in range(UNROLL):
                        _ln_row(data_v.at[b], r0 + u, g, bt)
                    return c2

                lax.fori_loop(0, chunk // UNROLL, rows_body, 0)
                for c in writeback_copies(ci, b):
                    c.start()

                @pl.when(gi < n_pairs - 1)
                def _():
                    for c in writeback_copies(ci, b):
                        c.wait()
                    gather(ci + NBUF, b).start()
            return carry

        lax.fori_loop(0, n_pairs, pair_body, 0)
        for b in range(NBUF):
            for c in writeback_copies(n_chunks - NBUF + b, b):
                c.wait()

    return k(table, idx, gamma, beta)


def kernel(input, table, gamma, beta):
    idx = input.reshape(-1).astype(jnp.int32)
    B = idx.shape[0]
    chunk = 400
    n_chunks = B // (32 * chunk)
    tlin = _transpose_table(table.T).reshape(-1, D)
    return _embed_ln(tlin, idx, gamma, beta, n_chunks, chunk,
                     input.shape[0], input.shape[1])


# speedup vs baseline: 1.1048x; 1.0080x over previous
import functools

import jax
import jax.numpy as jnp
from jax import lax
from jax.experimental import pallas as pl
from jax.experimental.pallas import tpu as pltpu
from jax.experimental.pallas import tpu_sc as plsc

D = 64
L = 16
EPS = 1e-5
NBUF = 2
UNROLL = 4


def _splat_sum(v):
    iota = lax.broadcasted_iota(jnp.int32, (L,), 0)
    for sh in (8, 4, 2, 1):
        v = v + v.at[iota ^ sh].get(mode="promise_in_bounds")
    return v


def _rsqrt(x):
    i = lax.bitcast_convert_type(x, jnp.int32)
    i = jnp.int32(0x5F3759DF) - (i >> 1)
    y = lax.bitcast_convert_type(i, jnp.float32)
    xh = x * 0.5
    y = y * (1.5 - xh * y * y)
    y = y * (1.5 - xh * y * y)
    return y


def _ln_row(data, r, g, bt):
    x = [data[r, pl.ds(j * L, L)] for j in range(D // L)]
    s = (x[0] + x[1]) + (x[2] + x[3])
    q = ((x[0] * x[0] + x[1] * x[1]) + (x[2] * x[2] + x[3] * x[3]))
    mean = _splat_sum(s) * (1.0 / D)
    ex2 = _splat_sum(q) * (1.0 / D) + EPS
    rstd = _rsqrt(ex2 - mean * mean)
    for j in range(D // L):
        data[r, pl.ds(j * L, L)] = (x[j] - mean) * rstd * g[j] + bt[j]


@jax.jit
def _transpose_table(table_t):
    V = table_t.shape[1]
    NC, NS = 2, 16
    NW = NC * NS
    n_blocks = (V + 127) // 128
    base_blocks = n_blocks // NW
    extra = n_blocks - base_blocks * NW

    mesh = plsc.VectorSubcoreMesh(core_axis_name="c", subcore_axis_name="s")

    @functools.partial(
        pl.kernel,
        mesh=mesh,
        out_type=jax.ShapeDtypeStruct((n_blocks * 128 * D,), jnp.float32),
        scratch_types=[
            pltpu.VMEM((D, 128), jnp.float32),
            pltpu.VMEM((D, 128), jnp.float32),
            pltpu.VMEM((128 * D,), jnp.float32),
            pltpu.VMEM((128 * D,), jnp.float32),
            pltpu.VMEM((128 * (D + 1),), jnp.float32),
            pltpu.SemaphoreType.DMA((NBUF,)),
            pltpu.SemaphoreType.DMA((NBUF,)),
        ],
        compiler_params=pltpu.CompilerParams(
            use_tc_tiling_on_sc=True, needs_layout_passes=False),
    )
    def k(tt_hbm, tlin_hbm, in_v0, in_v1, out_v0, out_v1, skew_v, isem, osem):
        in_v = [in_v0, in_v1]
        out_v = [out_v0, out_v1]
        wid = lax.axis_index("s") * NC + lax.axis_index("c")
        nblk = base_blocks + jnp.where(wid < extra, 1, 0)
        start = wid * base_blocks + jnp.minimum(wid, extra)

        def slab_in_copies(i, b):
            return [
                pltpu.make_async_copy(
                    tt_hbm.at[pl.ds(t * 8, 8), pl.ds((start + i) * 128, 128)],
                    in_v[b].at[pl.ds(t * 8, 8)], isem.at[b])
                for t in range(D // 8)
            ]

        def slab_in(i, b):
            class _G:
                def start(self):
                    for c in slab_in_copies(i, b):
                        c.start()

                def wait(self):
                    for c in slab_in_copies(i, b):
                        c.wait()
            return _G()

        def slab_out(i, b):
            return pltpu.make_async_copy(
                out_v[b],
                tlin_hbm.at[pl.ds((start + i) * 128 * D, 128 * D)],
                osem.at[b])

        S = D + 1
        iota = lax.broadcasted_iota(jnp.int32, (L,), 0)
        iota_skew = iota * S
        zeros = jnp.zeros((L,), jnp.int32)
        cols = [iota + j * L for j in range(8)]

        for b in range(NBUF):
            @pl.when(b < nblk)
            def _():
                slab_in(b, b).start()

        n_pairs = base_blocks // NBUF + 1

        def pair_body(g, carry):
            for b in range(NBUF):
                i = g * NBUF + b

                @pl.when(i < nblk)
                def _():
                    slab_in(i, b).wait()

                    @pl.when(i >= NBUF)
                    def _():
                        slab_out(i - NBUF, b).wait()

                    def f_body(f2, c2):
                        for u in range(4):
                            f = f2 * 4 + u
                            rowv = zeros + f
                            for j in range(8):
                                v = plsc.load_gather(in_v[b], [rowv, cols[j]])
                                addr = iota_skew + (j * L * S + f)
                                plsc.store_scatter(skew_v, [addr], v)
                        return c2

                    lax.fori_loop(0, D // 4, f_body, 0)

                    def e_body(e2, c2):
                        for u in range(4):
                            e = e2 * 4 + u
                            for j in range(D // L):
                                src = zeros + (e * S + j * L)
                                v = plsc.load_gather(skew_v, [src + iota])
                                out_v[b][pl.ds(e * D + j * L, L)] = v
                        return c2

                    lax.fori_loop(0, 32, e_body, 0)
                    slab_out(i, b).start()

                    @pl.when(i + NBUF < nblk)
                    def _():
                        slab_in(i + NBUF, b).start()
            return carry

        lax.fori_loop(0, n_pairs, pair_body, 0)
        for b in range(NBUF):
            last = jnp.where(lax.rem(nblk - 1, NBUF) == b, nblk - 1, nblk - 2)

            @pl.when(last >= 0)
            def _():
                slab_out(last, b).wait()

    return k(table_t)


@functools.partial(jax.jit, static_argnames=("n_chunks", "chunk", "out_rows", "out_cols"))
def _embed_ln(table, idx, gamma, beta, n_chunks, chunk, out_rows, out_cols):
    B = idx.shape[0]
    NC, NS = 2, 16
    NW = NC * NS
    b_per_w = B // NW
    n_pairs = n_chunks // NBUF
    nb = chunk // out_cols

    mesh = plsc.VectorSubcoreMesh(core_axis_name="c", subcore_axis_name="s")

    @functools.partial(
        pl.kernel,
        mesh=mesh,
        out_type=jax.ShapeDtypeStruct((out_rows, out_cols, D), jnp.float32),
        scratch_types=[
            pltpu.VMEM((b_per_w,), jnp.int32),
            pltpu.VMEM((NBUF, chunk, D), jnp.float32),
            pltpu.VMEM((D,), jnp.float32),
            pltpu.VMEM((D,), jnp.float32),
            pltpu.SemaphoreType.DMA((NBUF,)),
            pltpu.SemaphoreType.DMA((NBUF,)),
        ],
        compiler_params=pltpu.CompilerParams(use_tc_tiling_on_sc=False),
    )
    def k(table_hbm, idx_hbm, gamma_hbm, beta_hbm, out_hbm,
          idx_v, data_v, g_v, bt_v, gsem, osem):
        wid = lax.axis_index("s") * NC + lax.axis_index("c")
        base = wid * b_per_w

        pltpu.sync_copy(idx_hbm.at[pl.ds(base, b_per_w)], idx_v)
        pltpu.sync_copy(gamma_hbm, g_v)
        pltpu.sync_copy(beta_hbm, bt_v)
        g = [g_v[pl.ds(j * L, L)] for j in range(D // L)]
        bt = [bt_v[pl.ds(j * L, L)] for j in range(D // L)]

        def gather(ci, b):
            return pltpu.make_async_copy(
                table_hbm.at[idx_v.at[pl.ds(ci * chunk, chunk)]],
                data_v.at[b], gsem.at[b])

        def writeback_copies(ci, b):
            blk0 = (base + ci * chunk) // out_cols
            return [
                pltpu.make_async_copy(
                    data_v.at[b, pl.ds(j * out_cols, out_cols)],
                    out_hbm.at[blk0 + j], osem.at[b])
                for j in range(nb)
            ]

        for b in range(NBUF):
            gather(b, b).start()

        def pair_body(gi, carry):
            for b in range(NBUF):
                ci = gi * NBUF + b
                gather(ci, b).wait()

                def rows_body(t, c2):
                    r0 = t * UNROLL
                    for u in range(UNROLL):
                        _ln_row(data_v.at[b], r0 + u, g, bt)
                    return c2

                lax.fori_loop(0, chunk // UNROLL, rows_body, 0)
                for c in writeback_copies(ci, b):
                    c.start()

                @pl.when(gi < n_pairs - 1)
                def _():
                    for c in writeback_copies(ci, b):
                        c.wait()
                    gather(ci + NBUF, b).start()
            return carry

        lax.fori_loop(0, n_pairs, pair_body, 0)
        for b in range(NBUF):
            for c in writeback_copies(n_chunks - NBUF + b, b):
                c.wait()

    return k(table, idx, gamma, beta)


def kernel(input, table, gamma, beta):
    idx = input.reshape(-1).astype(jnp.int32)
    B = idx.shape[0]
    chunk = 400
    n_chunks = B // (32 * chunk)
    tlin = _transpose_table(table.T).reshape(-1, D)
    return _embed_ln(tlin, idx, gamma, beta, n_chunks, chunk,
                     input.shape[0], input.shape[1])


# speedup vs baseline: 1.4173x; 1.2829x over previous
import functools

import jax
import jax.numpy as jnp
from jax import lax
from jax.experimental import pallas as pl
from jax.experimental.pallas import tpu as pltpu
from jax.experimental.pallas import tpu_sc as plsc

D = 64
L = 16
EPS = 1e-5
NBUF = 2
UNROLL = 4


def _splat_sum(v):
    iota = lax.broadcasted_iota(jnp.int32, (L,), 0)
    for sh in (8, 4, 2, 1):
        v = v + v.at[iota ^ sh].get(mode="promise_in_bounds")
    return v


def _rsqrt(x):
    i = lax.bitcast_convert_type(x, jnp.int32)
    i = jnp.int32(0x5F3759DF) - (i >> 1)
    y = lax.bitcast_convert_type(i, jnp.float32)
    xh = x * 0.5
    y = y * (1.5 - xh * y * y)
    y = y * (1.5 - xh * y * y)
    return y


def _ln_row(data, r, g, bt):
    x = [data[r, pl.ds(j * L, L)] for j in range(D // L)]
    s = (x[0] + x[1]) + (x[2] + x[3])
    q = ((x[0] * x[0] + x[1] * x[1]) + (x[2] * x[2] + x[3] * x[3]))
    mean = _splat_sum(s) * (1.0 / D)
    ex2 = _splat_sum(q) * (1.0 / D) + EPS
    rstd = _rsqrt(ex2 - mean * mean)
    for j in range(D // L):
        data[r, pl.ds(j * L, L)] = (x[j] - mean) * rstd * g[j] + bt[j]


@functools.partial(jax.jit, static_argnames=("n_chunks", "chunk", "out_rows", "out_cols"))
def _embed_ln(table, idx, gamma, beta, n_chunks, chunk, out_rows, out_cols):
    B = idx.shape[0]
    NC, NS = 2, 16
    NW = NC * NS
    b_per_w = B // NW
    n_pairs = n_chunks // NBUF
    nb = chunk // out_cols

    mesh = plsc.VectorSubcoreMesh(core_axis_name="c", subcore_axis_name="s")

    @functools.partial(
        pl.kernel,
        mesh=mesh,
        out_type=jax.ShapeDtypeStruct((out_rows, out_cols, D), jnp.float32),
        scratch_types=[
            pltpu.VMEM((b_per_w,), jnp.int32),
            pltpu.VMEM((NBUF, chunk, D), jnp.float32),
            pltpu.VMEM((D,), jnp.float32),
            pltpu.VMEM((D,), jnp.float32),
            pltpu.SemaphoreType.DMA((NBUF,)),
            pltpu.SemaphoreType.DMA((NBUF,)),
        ],
        compiler_params=pltpu.CompilerParams(use_tc_tiling_on_sc=False),
    )
    def k(table_hbm, idx_hbm, gamma_hbm, beta_hbm, out_hbm,
          idx_v, data_v, g_v, bt_v, gsem, osem):
        wid = lax.axis_index("s") * NC + lax.axis_index("c")
        base = wid * b_per_w

        pltpu.sync_copy(idx_hbm.at[pl.ds(base, b_per_w)], idx_v)
        pltpu.sync_copy(gamma_hbm, g_v)
        pltpu.sync_copy(beta_hbm, bt_v)
        g = [g_v[pl.ds(j * L, L)] for j in range(D // L)]
        bt = [bt_v[pl.ds(j * L, L)] for j in range(D // L)]

        def gather(ci, b):
            return pltpu.make_async_copy(
                table_hbm.at[idx_v.at[pl.ds(ci * chunk, chunk)]],
                data_v.at[b], gsem.at[b])

        def writeback_copies(ci, b):
            blk0 = (base + ci * chunk) // out_cols
            return [
                pltpu.make_async_copy(
                    data_v.at[b, pl.ds(j * out_cols, out_cols)],
                    out_hbm.at[blk0 + j], osem.at[b])
                for j in range(nb)
            ]

        for b in range(NBUF):
            gather(b, b).start()

        def pair_body(gi, carry):
            for b in range(NBUF):
                ci = gi * NBUF + b
                gather(ci, b).wait()

                def rows_body(t, c2):
                    r0 = t * UNROLL
                    for u in range(UNROLL):
                        _ln_row(data_v.at[b], r0 + u, g, bt)
                    return c2

                lax.fori_loop(0, chunk // UNROLL, rows_body, 0)
                for c in writeback_copies(ci, b):
                    c.start()

                @pl.when(gi < n_pairs - 1)
                def _():
                    for c in writeback_copies(ci, b):
                        c.wait()
                    gather(ci + NBUF, b).start()
            return carry

        lax.fori_loop(0, n_pairs, pair_body, 0)
        for b in range(NBUF):
            for c in writeback_copies(n_chunks - NBUF + b, b):
                c.wait()

    return k(table, idx, gamma, beta)


def kernel(input, table, gamma, beta):
    idx = input.reshape(-1).astype(jnp.int32)
    B = idx.shape[0]
    chunk = 400
    n_chunks = B // (32 * chunk)
    return _embed_ln(table, idx, gamma, beta, n_chunks, chunk,
                     input.shape[0], input.shape[1])
